# Initial kernel scaffold; baseline (speedup 1.0000x reference)
import jax
import jax.numpy as jnp
from jax.experimental import pallas as pl


def kernel(indices, core0, core1, core2):
    raise NotImplementedError("write your pallas kernel here")


# speedup vs baseline: 5.9439x; 5.9439x over previous
import functools

import jax
import jax.numpy as jnp
from jax import lax
from jax.experimental import pallas as pl
from jax.experimental.pallas import tpu as pltpu
from jax.experimental.pallas import tpu_sc as plsc

B = 16384
F = 100
D0, D1, D2 = 64, 1024, 128

NC, NS, L = 2, 16, 16
NW = NC * NS
BPW = B // NW
C = 32
NCHUNK = BPW // C


def _body(t0, t1, t2, idx_hbm, o0, o1, o2,
          idxv, c0c, c1c, c2c, g0, g1, g2, gsem, ssem):
  wid = lax.axis_index("s") * NC + lax.axis_index("c")
  base = wid * BPW

  pltpu.sync_copy(idx_hbm.at[pl.ds(base, BPW)], idxv)

  fvec = jnp.full((L,), F, dtype=jnp.int32)

  @pl.loop(0, NCHUNK)
  def chunk(j):
    off = j * C
    for b in range(C // L):
      v = idxv[pl.ds(off + b * L, L)]
      c0 = lax.rem(v, fvec)
      r = lax.div(v, fvec)
      c1 = lax.rem(r, fvec)
      c2 = lax.div(r, fvec)
      c0c[pl.ds(b * L, L)] = c0
      c1c[pl.ds(b * L, L)] = c1
      c2c[pl.ds(b * L, L)] = c2
    ha = pltpu.async_copy(t0.at[c0c], g0, gsem)
    hb = pltpu.async_copy(t1.at[c1c], g1, gsem)
    hc = pltpu.async_copy(t2.at[c2c], g2, gsem)
    ha.wait()
    hb.wait()
    hc.wait()
    row = base + off
    sa = pltpu.async_copy(g0, o0.at[pl.ds(row, C)], ssem)
    sb = pltpu.async_copy(g1, o1.at[pl.ds(row, C)], ssem)
    sc = pltpu.async_copy(g2, o2.at[pl.ds(row, C)], ssem)
    sa.wait()
    sb.wait()
    sc.wait()


@jax.jit
def _run(t0, t1, t2, indices):
  mesh = plsc.VectorSubcoreMesh(core_axis_name="c", subcore_axis_name="s")
  fn = pl.kernel(
      _body,
      mesh=mesh,
      out_type=[
          jax.ShapeDtypeStruct((B, 128), jnp.float32),
          jax.ShapeDtypeStruct((B, D1), jnp.float32),
          jax.ShapeDtypeStruct((B, D2), jnp.float32),
      ],
      scratch_types=[
          pltpu.VMEM((BPW,), jnp.int32),
          pltpu.VMEM((C,), jnp.int32),
          pltpu.VMEM((C,), jnp.int32),
          pltpu.VMEM((C,), jnp.int32),
          pltpu.VMEM((C, 128), jnp.float32),
          pltpu.VMEM((C, D1), jnp.float32),
          pltpu.VMEM((C, D2), jnp.float32),
          pltpu.SemaphoreType.DMA,
          pltpu.SemaphoreType.DMA,
      ],
  )
  return fn(t0, t1, t2, indices)


def kernel(indices, core0, core1, core2):
  r0 = core0.shape[0]
  r1 = core1.shape[0]
  r2 = core2.shape[0]
  e0, e1, e2 = core0.shape[2], core1.shape[2], core2.shape[2]
  s0, s1, s2 = core0.shape[3], core1.shape[3], core2.shape[3]
  t0 = jnp.pad(jnp.transpose(core0, (1, 0, 2, 3)).reshape(F, D0),
               ((0, 0), (0, 128 - D0)))
  t1 = jnp.transpose(core1, (1, 0, 2, 3)).reshape(F, D1)
  t2 = jnp.transpose(core2, (1, 0, 2, 3)).reshape(F, D2)
  o0, o1, o2 = _run(t0, t1, t2, indices)
  return (
      o0[:, :D0].reshape(B, r0, e0, s0),
      o1.reshape(B, r1, e1, s1),
      o2.reshape(B, r2, e2, s2),
  )


# speedup vs baseline: 6.0986x; 1.0260x over previous
import functools

import jax
import jax.numpy as jnp
from jax import lax
from jax.experimental import pallas as pl
from jax.experimental.pallas import tpu as pltpu
from jax.experimental.pallas import tpu_sc as plsc

B = 16384
F = 100
D0, D1, D2 = 64, 1024, 128

NC, NS, L = 2, 16, 16
NW = NC * NS
BPW = B // NW
C = 32
NCHUNK = BPW // C


def _body(t0, t1, t2, idx_hbm, o0, o1, o2,
          idxv, c0c, c1c, c2c, g0, g1, g2, gsem, ssem):
  wid = lax.axis_index("s") * NC + lax.axis_index("c")
  base = wid * BPW

  pltpu.sync_copy(idx_hbm.at[pl.ds(base, BPW)], idxv)

  fvec = jnp.full((L,), F, dtype=jnp.int32)

  def digits(j, s):
    for b in range(C // L):
      v = idxv[pl.ds(j * C + b * L, L)]
      c0 = lax.rem(v, fvec)
      r = lax.div(v, fvec)
      c1 = lax.rem(r, fvec)
      c2 = lax.div(r, fvec)
      c0c[s, pl.ds(b * L, L)] = c0
      c1c[s, pl.ds(b * L, L)] = c1
      c2c[s, pl.ds(b * L, L)] = c2

  def gathers(s):
    a = pltpu.make_async_copy(t0.at[c0c.at[s]], g0.at[s], gsem)
    b = pltpu.make_async_copy(t1.at[c1c.at[s]], g1.at[s], gsem)
    c = pltpu.make_async_copy(t2.at[c2c.at[s]], g2.at[s], gsem)
    return (a, b, c)

  def scatters(j, s):
    row = base + j * C
    a = pltpu.make_async_copy(g0.at[s], o0.at[pl.ds(row, C)], ssem)
    b = pltpu.make_async_copy(g1.at[s], o1.at[pl.ds(row, C)], ssem)
    c = pltpu.make_async_copy(g2.at[s], o2.at[pl.ds(row, C)], ssem)
    return (a, b, c)

  digits(0, 0)
  for h in gathers(0):
    h.start()

  @pl.loop(0, NCHUNK)
  def chunk(j):
    s = j % 2
    ns = 1 - s

    @pl.when(j + 1 < NCHUNK)
    def _prefetch():
      digits(j + 1, ns)

      @pl.when(j >= 1)
      def _drain_prev_scatter():
        for h in scatters(j - 1, ns):
          h.wait()

      for h in gathers(ns):
        h.start()

    for h in gathers(s):
      h.wait()
    for h in scatters(j, s):
      h.start()

  for j in (NCHUNK - 2, NCHUNK - 1):
    for h in scatters(j, j % 2):
      h.wait()


@jax.jit
def _run(t0, t1, t2, indices):
  mesh = plsc.VectorSubcoreMesh(core_axis_name="c", subcore_axis_name="s")
  fn = pl.kernel(
      _body,
      mesh=mesh,
      out_type=[
          jax.ShapeDtypeStruct((B, 128), jnp.float32),
          jax.ShapeDtypeStruct((B, D1), jnp.float32),
          jax.ShapeDtypeStruct((B, D2), jnp.float32),
      ],
      scratch_types=[
          pltpu.VMEM((BPW,), jnp.int32),
          pltpu.VMEM((2, C), jnp.int32),
          pltpu.VMEM((2, C), jnp.int32),
          pltpu.VMEM((2, C), jnp.int32),
          pltpu.VMEM((2, C, 128), jnp.float32),
          pltpu.VMEM((2, C, D1), jnp.float32),
          pltpu.VMEM((2, C, D2), jnp.float32),
          pltpu.SemaphoreType.DMA,
          pltpu.SemaphoreType.DMA,
      ],
  )
  return fn(t0, t1, t2, indices)


def kernel(indices, core0, core1, core2):
  r0 = core0.shape[0]
  r1 = core1.shape[0]
  r2 = core2.shape[0]
  e0, e1, e2 = core0.shape[2], core1.shape[2], core2.shape[2]
  s0, s1, s2 = core0.shape[3], core1.shape[3], core2.shape[3]
  t0 = jnp.pad(jnp.transpose(core0, (1, 0, 2, 3)).reshape(F, D0),
               ((0, 0), (0, 128 - D0)))
  t1 = jnp.transpose(core1, (1, 0, 2, 3)).reshape(F, D1)
  t2 = jnp.transpose(core2, (1, 0, 2, 3)).reshape(F, D2)
  o0, o1, o2 = _run(t0, t1, t2, indices)
  return (
      o0[:, :D0].reshape(B, r0, e0, s0),
      o1.reshape(B, r1, e1, s1),
      o2.reshape(B, r2, e2, s2),
  )


# speedup vs baseline: 11.1074x; 1.8213x over previous
import functools

import jax
import jax.numpy as jnp
from jax import lax
from jax.experimental import pallas as pl
from jax.experimental.pallas import tpu as pltpu
from jax.experimental.pallas import tpu_sc as plsc

B = 16384
F = 100
D0, D1, D2 = 64, 1024, 128

NC, NS, L = 2, 16, 16
NW = NC * NS
BPW = B // NW
C = 32
NCHUNK = BPW // C

BN = 512
KP = 128


def _sc_body(t0, t2, idx_hbm, o0, o2,
             idxv, c0c, c2c, g0, g2, gsem, ssem):
  wid = lax.axis_index("s") * NC + lax.axis_index("c")
  base = wid * BPW

  pltpu.sync_copy(idx_hbm.at[pl.ds(base, BPW)], idxv)

  fvec = jnp.full((L,), F, dtype=jnp.int32)
  f2vec = jnp.full((L,), F * F, dtype=jnp.int32)

  def digits(j, s):
    for b in range(C // L):
      v = idxv[pl.ds(j * C + b * L, L)]
      c0c[s, pl.ds(b * L, L)] = lax.rem(v, fvec)
      c2c[s, pl.ds(b * L, L)] = lax.div(v, f2vec)

  def gathers(s):
    a = pltpu.make_async_copy(t0.at[c0c.at[s]], g0.at[s], gsem)
    c = pltpu.make_async_copy(t2.at[c2c.at[s]], g2.at[s], gsem)
    return (a, c)

  def scatters(j, s):
    row = base + j * C
    a = pltpu.make_async_copy(g0.at[s], o0.at[pl.ds(row, C)], ssem)
    c = pltpu.make_async_copy(g2.at[s], o2.at[pl.ds(row, C)], ssem)
    return (a, c)

  digits(0, 0)
  for h in gathers(0):
    h.start()

  @pl.loop(0, NCHUNK)
  def chunk(j):
    s = j % 2
    ns = 1 - s

    @pl.when(j + 1 < NCHUNK)
    def _prefetch():
      digits(j + 1, ns)

      @pl.when(j >= 1)
      def _drain_prev_scatter():
        for h in scatters(j - 1, ns):
          h.wait()

      for h in gathers(ns):
        h.start()

    for h in gathers(s):
      h.wait()
    for h in scatters(j, s):
      h.start()

  for j in (NCHUNK - 2, NCHUNK - 1):
    for h in scatters(j, j % 2):
      h.wait()


def _tc_body(idx_ref, t_ref, o_ref):
  idxb = idx_ref[0, 0, :]
  c1 = lax.rem(lax.div(idxb, F), F)
  sel = jax.lax.broadcasted_iota(jnp.int32, (KP, BN), 0) == c1[None, :]
  onehot = sel.astype(jnp.float32)
  o_ref[...] = jnp.dot(t_ref[...], onehot,
                       preferred_element_type=jnp.float32)


@jax.jit
def _run(t0, t2, t1t, indices):
  mesh = plsc.VectorSubcoreMesh(core_axis_name="c", subcore_axis_name="s")
  sc_fn = pl.kernel(
      _sc_body,
      mesh=mesh,
      out_type=[
          jax.ShapeDtypeStruct((B, 128), jnp.float32),
          jax.ShapeDtypeStruct((B, D2), jnp.float32),
      ],
      scratch_types=[
          pltpu.VMEM((BPW,), jnp.int32),
          pltpu.VMEM((2, C), jnp.int32),
          pltpu.VMEM((2, C), jnp.int32),
          pltpu.VMEM((2, C, 128), jnp.float32),
          pltpu.VMEM((2, C, D2), jnp.float32),
          pltpu.SemaphoreType.DMA,
          pltpu.SemaphoreType.DMA,
      ],
  )
  o0, o2 = sc_fn(t0, t2, indices)

  o1t = pl.pallas_call(
      _tc_body,
      grid=(B // BN,),
      in_specs=[
          pl.BlockSpec((1, 1, BN), lambda n: (n, 0, 0)),
          pl.BlockSpec((D1, KP), lambda n: (0, 0)),
      ],
      out_specs=pl.BlockSpec((D1, BN), lambda n: (0, n)),
      out_shape=jax.ShapeDtypeStruct((D1, B), jnp.float32),
  )(indices.reshape(B // BN, 1, BN), t1t)
  return o0, o1t, o2


def kernel(indices, core0, core1, core2):
  r0 = core0.shape[0]
  r1 = core1.shape[0]
  r2 = core2.shape[0]
  e0, e1, e2 = core0.shape[2], core1.shape[2], core2.shape[2]
  s0, s1, s2 = core0.shape[3], core1.shape[3], core2.shape[3]
  t0 = jnp.pad(jnp.transpose(core0, (1, 0, 2, 3)).reshape(F, D0),
               ((0, 0), (0, 128 - D0)))
  t2 = jnp.transpose(core2, (1, 0, 2, 3)).reshape(F, D2)
  t1t = jnp.pad(core1.reshape(r1, F, e1 * s1).transpose(0, 2, 1)
                .reshape(D1, F), ((0, 0), (0, KP - F)))
  o0, o1t, o2 = _run(t0, t2, t1t, indices)
  out1 = o1t.reshape(r1, e1, s1, B).transpose(3, 0, 1, 2)
  return (
      o0[:, :D0].reshape(B, r0, e0, s0),
      out1,
      o2.reshape(B, r2, e2, s2),
  )


# speedup vs baseline: 11.1742x; 1.0060x over previous
import functools

import jax
import jax.numpy as jnp
from jax import lax
from jax.experimental import pallas as pl
from jax.experimental.pallas import tpu as pltpu
from jax.experimental.pallas import tpu_sc as plsc

B = 16384
F = 100
D0, D1, D2 = 64, 1024, 128

NC, NS, L = 2, 16, 16
NW = NC * NS
BPW = B // NW
C = 32
NCHUNK = BPW // C

BN = 512
KP = 128


def _sc_body(t0, t2, idx_hbm, o0, o2,
             idxv, c0c, c2c, g0, g2, gsem, ssem):
  wid = lax.axis_index("s") * NC + lax.axis_index("c")
  base = wid * BPW

  pltpu.sync_copy(idx_hbm.at[pl.ds(base, BPW)], idxv)

  fvec = jnp.full((L,), F, dtype=jnp.int32)
  f2vec = jnp.full((L,), F * F, dtype=jnp.int32)

  def digits(j, s):
    for b in range(C // L):
      v = idxv[pl.ds(j * C + b * L, L)]
      c0c[s, pl.ds(b * L, L)] = lax.rem(v, fvec)
      c2c[s, pl.ds(b * L, L)] = lax.div(v, f2vec)

  def gathers(s):
    a = pltpu.make_async_copy(t0.at[c0c.at[s]], g0.at[s], gsem)
    c = pltpu.make_async_copy(t2.at[c2c.at[s]], g2.at[s], gsem)
    return (a, c)

  def scatters(j, s):
    row = base + j * C
    a = pltpu.make_async_copy(g0.at[s], o0.at[pl.ds(row, C)], ssem)
    c = pltpu.make_async_copy(g2.at[s], o2.at[pl.ds(row, C)], ssem)
    return (a, c)

  digits(0, 0)
  for h in gathers(0):
    h.start()

  @pl.loop(0, NCHUNK)
  def chunk(j):
    s = j % 2
    ns = 1 - s

    @pl.when(j + 1 < NCHUNK)
    def _prefetch():
      digits(j + 1, ns)

      @pl.when(j >= 1)
      def _drain_prev_scatter():
        for h in scatters(j - 1, ns):
          h.wait()

      for h in gathers(ns):
        h.start()

    for h in gathers(s):
      h.wait()
    for h in scatters(j, s):
      h.start()

  for j in (NCHUNK - 2, NCHUNK - 1):
    for h in scatters(j, j % 2):
      h.wait()


def _tc_body(idx_ref, t_ref, o_ref):
  idxb = idx_ref[0, 0, :]
  c1 = lax.rem(lax.div(idxb, F), F)
  sel = jax.lax.broadcasted_iota(jnp.int32, (KP, BN), 0) == c1[None, :]
  onehot = sel.astype(jnp.float32)
  o_ref[...] = jnp.dot(t_ref[...], onehot,
                       preferred_element_type=jnp.float32)


@jax.jit
def _run(t0, t2, t1t, indices):
  mesh = plsc.VectorSubcoreMesh(core_axis_name="c", subcore_axis_name="s")
  sc_fn = pl.kernel(
      _sc_body,
      mesh=mesh,
      out_type=[
          jax.ShapeDtypeStruct((B, 128), jnp.float32),
          jax.ShapeDtypeStruct((B, D2), jnp.float32),
      ],
      scratch_types=[
          pltpu.VMEM((BPW,), jnp.int32),
          pltpu.VMEM((2, C), jnp.int32),
          pltpu.VMEM((2, C), jnp.int32),
          pltpu.VMEM((2, C, 128), jnp.float32),
          pltpu.VMEM((2, C, D2), jnp.float32),
          pltpu.SemaphoreType.DMA,
          pltpu.SemaphoreType.DMA,
      ],
  )
  o1t = pl.pallas_call(
      _tc_body,
      grid=(B // BN,),
      in_specs=[
          pl.BlockSpec((1, 1, BN), lambda n: (n, 0, 0)),
          pl.BlockSpec((D1, KP), lambda n: (0, 0)),
      ],
      out_specs=pl.BlockSpec((D1, BN), lambda n: (0, n)),
      out_shape=jax.ShapeDtypeStruct((D1, B), jnp.float32),
  )(indices.reshape(B // BN, 1, BN), t1t)

  o0, o2 = sc_fn(t0, t2, indices)
  return o0, o1t, o2


def kernel(indices, core0, core1, core2):
  r0 = core0.shape[0]
  r1 = core1.shape[0]
  r2 = core2.shape[0]
  e0, e1, e2 = core0.shape[2], core1.shape[2], core2.shape[2]
  s0, s1, s2 = core0.shape[3], core1.shape[3], core2.shape[3]
  t0 = jnp.pad(jnp.transpose(core0, (1, 0, 2, 3)).reshape(F, D0),
               ((0, 0), (0, 128 - D0)))
  t2 = jnp.transpose(core2, (1, 0, 2, 3)).reshape(F, D2)
  t1t = jnp.pad(core1.reshape(r1, F, e1 * s1).transpose(0, 2, 1)
                .reshape(D1, F), ((0, 0), (0, KP - F)))
  o0, o1t, o2 = _run(t0, t2, t1t, indices)
  out1 = o1t.reshape(r1, e1, s1, B).transpose(3, 0, 1, 2)
  return (
      o0[:, :D0].reshape(B, r0, e0, s0),
      out1,
      o2.reshape(B, r2, e2, s2),
  )


# speedup vs baseline: 16.0649x; 1.4377x over previous
import functools

import jax
import jax.numpy as jnp
from jax import lax
from jax.experimental import pallas as pl
from jax.experimental.pallas import tpu as pltpu
from jax.experimental.pallas import tpu_sc as plsc

B = 16384
F = 100
D0, D1, D2 = 64, 1024, 128

NC, NS, L = 2, 16, 16
NW = NC * NS
BPW = B // NW
C = 32
NCHUNK = BPW // C

BN = 1024
KP = 128


def _sc_body(t0, idx_hbm, o0, idxv, c0c, g0, gsem, ssem):
  wid = lax.axis_index("s") * NC + lax.axis_index("c")
  base = wid * BPW

  pltpu.sync_copy(idx_hbm.at[pl.ds(base, BPW)], idxv)

  fvec = jnp.full((L,), F, dtype=jnp.int32)

  def digits(j, s):
    for b in range(C // L):
      v = idxv[pl.ds(j * C + b * L, L)]
      c0c[s, pl.ds(b * L, L)] = lax.rem(v, fvec)

  def gather(s):
    return pltpu.make_async_copy(t0.at[c0c.at[s]], g0.at[s], gsem)

  def scatter(j, s):
    row = base + j * C
    return pltpu.make_async_copy(g0.at[s], o0.at[pl.ds(row, C)], ssem)

  digits(0, 0)
  gather(0).start()

  @pl.loop(0, NCHUNK)
  def chunk(j):
    s = j % 2
    ns = 1 - s

    @pl.when(j + 1 < NCHUNK)
    def _prefetch():
      digits(j + 1, ns)

      @pl.when(j >= 1)
      def _drain_prev_scatter():
        scatter(j - 1, ns).wait()

      gather(ns).start()

    gather(s).wait()
    scatter(j, s).start()

  scatter(NCHUNK - 2, 0).wait()
  scatter(NCHUNK - 1, 1).wait()


def _tc_body(idx_ref, t1_ref, t2_ref, o1_ref, o2_ref):
  idxb = idx_ref[0, 0, :]
  r = lax.div(idxb, F)
  c1 = lax.rem(r, F)
  c2 = lax.div(r, F)
  rows = jax.lax.broadcasted_iota(jnp.int32, (KP, BN), 0)
  oh1 = (rows == c1[None, :]).astype(jnp.float32)
  oh2 = (rows == c2[None, :]).astype(jnp.float32)
  o1_ref[...] = jnp.dot(t1_ref[...], oh1,
                        preferred_element_type=jnp.float32)
  r2 = jnp.dot(t2_ref[...], oh2, preferred_element_type=jnp.float32)
  o2_ref[...] = r2.reshape(D2, BN // 128, 128)


@jax.jit
def _run(t0, t1t, t2t, indices):
  mesh = plsc.VectorSubcoreMesh(core_axis_name="c", subcore_axis_name="s")
  sc_fn = pl.kernel(
      _sc_body,
      mesh=mesh,
      out_type=[
          jax.ShapeDtypeStruct((B, 128), jnp.float32),
      ],
      scratch_types=[
          pltpu.VMEM((BPW,), jnp.int32),
          pltpu.VMEM((2, C), jnp.int32),
          pltpu.VMEM((2, C, 128), jnp.float32),
          pltpu.SemaphoreType.DMA,
          pltpu.SemaphoreType.DMA,
      ],
  )

  o1t, o2t3 = pl.pallas_call(
      _tc_body,
      grid=(B // BN,),
      in_specs=[
          pl.BlockSpec((1, 1, BN), lambda n: (n, 0, 0)),
          pl.BlockSpec((D1, KP), lambda n: (0, 0)),
          pl.BlockSpec((D2, KP), lambda n: (0, 0)),
      ],
      out_specs=[
          pl.BlockSpec((D1, BN), lambda n: (0, n)),
          pl.BlockSpec((D2, BN // 128, 128), lambda n: (0, n, 0)),
      ],
      out_shape=[
          jax.ShapeDtypeStruct((D1, B), jnp.float32),
          jax.ShapeDtypeStruct((D2, B // 128, 128), jnp.float32),
      ],
  )(indices.reshape(B // BN, 1, BN), t1t, t2t)

  (o0,) = sc_fn(t0, indices)
  return o0, o1t, o2t3


def kernel(indices, core0, core1, core2):
  r0 = core0.shape[0]
  r1 = core1.shape[0]
  r2 = core2.shape[0]
  e0, e1, e2 = core0.shape[2], core1.shape[2], core2.shape[2]
  s0, s1, s2 = core0.shape[3], core1.shape[3], core2.shape[3]
  t0 = jnp.pad(jnp.transpose(core0, (1, 0, 2, 3)).reshape(F, D0),
               ((0, 0), (0, 128 - D0)))
  t1t = jnp.pad(core1.reshape(r1, F, e1 * s1).transpose(0, 2, 1)
                .reshape(D1, F), ((0, 0), (0, KP - F)))
  t2t = jnp.pad(core2.reshape(r2, F, e2 * s2).transpose(0, 2, 1)
                .reshape(D2, F), ((0, 0), (0, KP - F)))
  o0, o1t, o2t3 = _run(t0, t1t, t2t, indices)
  out1 = o1t.reshape(r1, e1, s1, B).transpose(3, 0, 1, 2)
  out2 = o2t3.reshape(r2, e2, s2, B).transpose(3, 0, 1, 2)
  return (
      o0[:, :D0].reshape(B, r0, e0, s0),
      out1,
      out2,
  )
